# baseline (speedup 1.0000x reference)
import functools

import jax
import jax.numpy as jnp
from jax import lax
from jax.experimental import pallas as pl
from jax.experimental.pallas import tpu as pltpu
from jax.experimental.pallas import tpu_sc as plsc

N_NODES = 100000
D = 256
N_SEG = 256

NUM_CORES = 2
NUM_SUBCORES = 16
NW = NUM_CORES * NUM_SUBCORES
SEG_PER_W = N_SEG // NW
LANES = 16
NVREG = D // LANES
CHUNK = 224


def _sc_body(feat_hbm, starts_hbm, out_hbm, starts_smem, buf0, buf1, acc, sem0, sem1):
  wid = lax.axis_index("s") * NUM_CORES + lax.axis_index("c")
  seg0 = wid * SEG_PER_W

  pltpu.sync_copy(starts_hbm.at[pl.ds(seg0, 16)], starts_smem)
  sv = starts_smem[...]

  neg_inf = jnp.full((LANES,), -jnp.inf, jnp.float32)
  for s in range(SEG_PER_W):
    for f in range(NVREG):
      acc[s, pl.ds(f * LANES, LANES)] = neg_inf

  row_start = sv[0]
  row_end = sv[SEG_PER_W]
  base = (row_start // 8) * 8
  n_chunks = (row_end - base + CHUNK - 1) // CHUNK

  def chunk_off(i):
    return pl.multiple_of(jnp.minimum(base + i * CHUNK, N_NODES - CHUNK), 8)

  bufs = (buf0, buf1)
  sems = (sem0, sem1)

  def start_load(i, slot):
    pltpu.async_copy(feat_hbm.at[pl.ds(chunk_off(i), CHUNK)], bufs[slot], sems[slot])

  def wait_load(slot):
    pltpu.make_async_copy(
        feat_hbm.at[pl.ds(0, CHUNK)], bufs[slot], sems[slot]).wait()

  @pl.when(n_chunks > 0)
  def _prime():
    start_load(0, 0)

  def process(i, slot):
    o = chunk_off(i)
    buf = bufs[slot]
    for s in range(SEG_PER_W):
      lo = jnp.maximum(sv[s] - o, 0)
      hi = jnp.minimum(sv[s + 1] - o, CHUNK)
      hi = jnp.maximum(hi, lo)

      def row_body(r, a):
        return tuple(
            jnp.maximum(a[f], buf[r, pl.ds(f * LANES, LANES)])
            for f in range(NVREG)
        )

      a0 = tuple(acc[s, pl.ds(f * LANES, LANES)] for f in range(NVREG))
      a1 = plsc.parallel_loop(lo, hi, 1, unroll=4, carry=a0)(row_body)
      for f in range(NVREG):
        acc[s, pl.ds(f * LANES, LANES)] = a1[f]

  def outer(i2, _):
    for b in range(2):
      i = i2 * 2 + b

      @pl.when(i < n_chunks)
      def _():
        nxt = i + 1

        @pl.when(nxt < n_chunks)
        def _():
          start_load(nxt, (b + 1) % 2)

        wait_load(b)
        process(i, b)

    return 0

  lax.fori_loop(0, (n_chunks + 1) // 2, outer, 0)

  pltpu.sync_copy(acc, out_hbm.at[pl.ds(seg0, SEG_PER_W)])


@jax.jit
def kernel(feat, segment_ids):
  edges = jnp.arange(N_SEG + 1, dtype=jnp.int32)
  starts = jnp.searchsorted(segment_ids, edges, side="left").astype(jnp.int32)
  starts = jnp.concatenate(
      [starts, jnp.full((7,), N_NODES, jnp.int32)])

  mesh = plsc.VectorSubcoreMesh(
      core_axis_name="c", subcore_axis_name="s",
      num_cores=NUM_CORES, num_subcores=NUM_SUBCORES)

  f = pl.kernel(
      _sc_body,
      out_type=jax.ShapeDtypeStruct((N_SEG, D), jnp.float32),
      mesh=mesh,
      scratch_types=[
          pltpu.VMEM((16,), jnp.int32),
          pltpu.VMEM((CHUNK, D), jnp.float32),
          pltpu.VMEM((CHUNK, D), jnp.float32),
          pltpu.VMEM((SEG_PER_W, D), jnp.float32),
          pltpu.SemaphoreType.DMA,
          pltpu.SemaphoreType.DMA,
      ],
  )
  return f(feat, starts)


# speedup vs baseline: 1.6533x; 1.6533x over previous
import functools

import jax
import jax.numpy as jnp
from jax import lax
from jax.experimental import pallas as pl
from jax.experimental.pallas import tpu as pltpu
from jax.experimental.pallas import tpu_sc as plsc

N_NODES = 100000
D = 256
N_SEG = 256

NUM_CORES = 2
NUM_SUBCORES = 16
NW = NUM_CORES * NUM_SUBCORES
SEG_PER_W = N_SEG // NW
LANES = 16
NVREG = D // LANES
CHUNK = 224


def _sc_body(feat_hbm, starts_hbm, out_hbm, starts_smem, buf0, buf1, acc, sem0, sem1):
  wid = lax.axis_index("s") * NUM_CORES + lax.axis_index("c")
  seg0 = wid * SEG_PER_W

  pltpu.sync_copy(starts_hbm.at[pl.ds(seg0, 16)], starts_smem)
  sv = starts_smem[...]

  neg_inf = jnp.full((LANES,), -jnp.inf, jnp.float32)
  for s in range(SEG_PER_W):
    for f in range(NVREG):
      acc[s, pl.ds(f * LANES, LANES)] = neg_inf

  row_start = sv[0]
  row_end = sv[SEG_PER_W]
  base = (row_start // 8) * 8
  n_chunks = (row_end - base + CHUNK - 1) // CHUNK

  def chunk_off(i):
    return pl.multiple_of(jnp.minimum(base + i * CHUNK, N_NODES - CHUNK), 8)

  bufs = (buf0, buf1)
  sems = (sem0, sem1)

  def start_load(i, slot):
    pltpu.async_copy(feat_hbm.at[pl.ds(chunk_off(i), CHUNK)], bufs[slot], sems[slot])

  def wait_load(slot):
    pltpu.make_async_copy(
        feat_hbm.at[pl.ds(0, CHUNK)], bufs[slot], sems[slot]).wait()

  @pl.when(n_chunks > 0)
  def _prime():
    start_load(0, 0)

  def process(i, slot):
    o = chunk_off(i)
    buf = bufs[slot]
    for s in range(SEG_PER_W):
      lo = jnp.maximum(sv[s] - o, 0)
      hi = jnp.minimum(sv[s + 1] - o, CHUNK)
      hi = jnp.maximum(hi, lo)

      def row_body(r, a):
        return tuple(
            jnp.maximum(a[f], buf[r, pl.ds(f * LANES, LANES)])
            for f in range(NVREG)
        )

      a0 = tuple(acc[s, pl.ds(f * LANES, LANES)] for f in range(NVREG))
      a1 = plsc.parallel_loop(lo, hi, 1, unroll=4, carry=a0)(row_body)
      for f in range(NVREG):
        acc[s, pl.ds(f * LANES, LANES)] = a1[f]

  def outer(i2, _):
    for b in range(2):
      i = i2 * 2 + b

      @pl.when(i < n_chunks)
      def _():
        nxt = i + 1

        @pl.when(nxt < n_chunks)
        def _():
          start_load(nxt, (b + 1) % 2)

        wait_load(b)
        process(i, b)

    return 0

  lax.fori_loop(0, (n_chunks + 1) // 2, outer, 0)

  pltpu.sync_copy(acc, out_hbm.at[pl.ds(seg0, SEG_PER_W)])


@jax.jit
def kernel(feat, segment_ids):
  edges = jnp.arange(N_SEG + 1, dtype=jnp.int32)
  starts = jnp.sum(
      (segment_ids[None, :] < edges[:, None]).astype(jnp.int32), axis=1)
  starts = jnp.concatenate(
      [starts, jnp.full((7,), N_NODES, jnp.int32)])

  mesh = plsc.VectorSubcoreMesh(
      core_axis_name="c", subcore_axis_name="s",
      num_cores=NUM_CORES, num_subcores=NUM_SUBCORES)

  f = pl.kernel(
      _sc_body,
      out_type=jax.ShapeDtypeStruct((N_SEG, D), jnp.float32),
      mesh=mesh,
      scratch_types=[
          pltpu.VMEM((16,), jnp.int32),
          pltpu.VMEM((CHUNK, D), jnp.float32),
          pltpu.VMEM((CHUNK, D), jnp.float32),
          pltpu.VMEM((SEG_PER_W, D), jnp.float32),
          pltpu.SemaphoreType.DMA,
          pltpu.SemaphoreType.DMA,
      ],
  )
  return f(feat, starts)


# speedup vs baseline: 1.9264x; 1.1652x over previous
import functools

import jax
import jax.numpy as jnp
from jax import lax
from jax.experimental import pallas as pl
from jax.experimental.pallas import tpu as pltpu
from jax.experimental.pallas import tpu_sc as plsc

N_NODES = 100000
D = 256
N_SEG = 256

NUM_CORES = 2
NUM_SUBCORES = 16
NW = NUM_CORES * NUM_SUBCORES
SEG_PER_W = N_SEG // NW
LANES = 16
NVREG = D // LANES
CHUNK = 224


def _sc_body(feat_hbm, starts_hbm, out_hbm, starts_smem, buf0, buf1, acc, sem0, sem1):
  wid = lax.axis_index("s") * NUM_CORES + lax.axis_index("c")
  seg0 = wid * SEG_PER_W

  pltpu.sync_copy(starts_hbm.at[pl.ds(seg0, 16)], starts_smem)
  sv = starts_smem[...]

  neg_inf = jnp.full((LANES,), -jnp.inf, jnp.float32)
  for s in range(SEG_PER_W):
    for f in range(NVREG):
      acc[s, pl.ds(f * LANES, LANES)] = neg_inf

  row_start = sv[0]
  row_end = sv[SEG_PER_W]
  base = (row_start // 8) * 8
  n_chunks = (row_end - base + CHUNK - 1) // CHUNK

  def chunk_off(i):
    return pl.multiple_of(jnp.minimum(base + i * CHUNK, N_NODES - CHUNK), 8)

  bufs = (buf0, buf1)
  sems = (sem0, sem1)

  def start_load(i, slot):
    pltpu.async_copy(feat_hbm.at[pl.ds(chunk_off(i), CHUNK)], bufs[slot], sems[slot])

  def wait_load(slot):
    pltpu.make_async_copy(
        feat_hbm.at[pl.ds(0, CHUNK)], bufs[slot], sems[slot]).wait()

  @pl.when(n_chunks > 0)
  def _prime():
    start_load(0, 0)

  def process(i, slot):
    o = chunk_off(i)
    buf = bufs[slot]
    for s in range(SEG_PER_W):
      lo = jnp.maximum(sv[s] - o, 0)
      hi = jnp.minimum(sv[s + 1] - o, CHUNK)
      hi = jnp.maximum(hi, lo)

      def row_body(r, a):
        return tuple(
            jnp.maximum(a[f], buf[r, pl.ds(f * LANES, LANES)])
            for f in range(NVREG)
        )

      a0 = tuple(acc[s, pl.ds(f * LANES, LANES)] for f in range(NVREG))
      a1 = plsc.parallel_loop(lo, hi, 1, unroll=4, carry=a0)(row_body)
      for f in range(NVREG):
        acc[s, pl.ds(f * LANES, LANES)] = a1[f]

  def outer(i2, _):
    for b in range(2):
      i = i2 * 2 + b

      @pl.when(i < n_chunks)
      def _():
        nxt = i + 1

        @pl.when(nxt < n_chunks)
        def _():
          start_load(nxt, (b + 1) % 2)

        wait_load(b)
        process(i, b)

    return 0

  lax.fori_loop(0, (n_chunks + 1) // 2, outer, 0)

  pltpu.sync_copy(acc, out_hbm.at[pl.ds(seg0, SEG_PER_W)])


@jax.jit
def kernel(feat, segment_ids):
  edges = jnp.arange(N_SEG + 1, dtype=jnp.int32)
  wnd = 500
  n_wnd = N_NODES // wnd
  windows = segment_ids.reshape(n_wnd, wnd)
  coarse = jnp.sum((windows[:, -1][None, :] < edges[:, None]).astype(jnp.int32),
                   axis=1)
  wclip = jnp.minimum(coarse, n_wnd - 1)
  brows = windows[wclip]
  inner = jnp.sum((brows < edges[:, None]).astype(jnp.int32), axis=1)
  starts = wclip * wnd + inner
  starts = jnp.concatenate(
      [starts, jnp.full((7,), N_NODES, jnp.int32)])

  mesh = plsc.VectorSubcoreMesh(
      core_axis_name="c", subcore_axis_name="s",
      num_cores=NUM_CORES, num_subcores=NUM_SUBCORES)

  f = pl.kernel(
      _sc_body,
      out_type=jax.ShapeDtypeStruct((N_SEG, D), jnp.float32),
      mesh=mesh,
      scratch_types=[
          pltpu.VMEM((16,), jnp.int32),
          pltpu.VMEM((CHUNK, D), jnp.float32),
          pltpu.VMEM((CHUNK, D), jnp.float32),
          pltpu.VMEM((SEG_PER_W, D), jnp.float32),
          pltpu.SemaphoreType.DMA,
          pltpu.SemaphoreType.DMA,
      ],
  )
  return f(feat, starts)
